# Initial kernel scaffold; baseline (speedup 1.0000x reference)
import jax
import jax.numpy as jnp
from jax.experimental import pallas as pl


def kernel(x, W_route, b_route, W_noise, b_noise, W1, b1, W2, b2):
    raise NotImplementedError("write your pallas kernel here")


# speedup vs baseline: 1.2371x; 1.2371x over previous
import jax
import jax.numpy as jnp
from jax import lax
from jax.experimental import pallas as pl
from jax.experimental.pallas import tpu as pltpu
from jax.experimental.pallas import tpu_sc as plsc

_T = 2048
_D = 768
_E = 8
_H = 3072
_TILE = 256
_NT = 24
_NTP = 32
_P = _NT * _TILE
_WSC = 128
_F = 2
_DV = _D // _F


def _router_body(x_ref, wr_ref, br_ref, pos_ref, wv_ref, te_ref, vf_ref):
    x = x_ref[...]
    logits = lax.dot_general(
        x, wr_ref[...], (((1,), (0,)), ((), ())),
        preferred_element_type=jnp.float32) + br_ref[...]
    ie = lax.broadcasted_iota(jnp.int32, (_T, _E), 1)

    v1 = jnp.max(logits, axis=1, keepdims=True)
    i1 = jnp.min(jnp.where(logits == v1, ie, _E), axis=1, keepdims=True)
    ch1 = ie == i1
    l2 = jnp.where(ch1, -jnp.inf, logits)
    v2 = jnp.max(l2, axis=1, keepdims=True)
    i2 = jnp.min(jnp.where(l2 == v2, ie, _E), axis=1, keepdims=True)
    ch2 = ie == i2

    e2 = jnp.exp(v2 - v1)
    denom = 1.0 + e2
    w1 = 1.0 / denom
    w2 = e2 / denom

    m = ch1.astype(jnp.int32) + ch2.astype(jnp.int32)
    cs = m
    s = 1
    while s < _T:
        cs = cs + jnp.concatenate(
            [jnp.zeros((s, _E), jnp.int32), cs[:-s]], axis=0)
        s *= 2
    counts = cs[_T - 1:_T, :]
    tilecnt = (counts + _TILE - 1) // _TILE
    incl = tilecnt
    s = 1
    while s < _E:
        incl = incl + jnp.concatenate(
            [jnp.zeros((1, s), jnp.int32), incl[:, :-s]], axis=1)
        s *= 2
    excl = incl - tilecnt
    start = excl * _TILE
    total = incl[:, _E - 1:_E]

    cse = cs - m
    startb = jnp.broadcast_to(start, (_T, _E))
    p1 = (jnp.sum(jnp.where(ch1, cse, 0), axis=1, keepdims=True)
          + jnp.sum(jnp.where(ch1, startb, 0), axis=1, keepdims=True))
    p2 = (jnp.sum(jnp.where(ch2, cse, 0), axis=1, keepdims=True)
          + jnp.sum(jnp.where(ch2, startb, 0), axis=1, keepdims=True))
    pos_ref[...] = jnp.concatenate([p1, p2], axis=1)
    wv_ref[...] = jnp.concatenate([w1, w2], axis=1)

    j = lax.broadcasted_iota(jnp.int32, (_NTP, _E), 0)
    eb = lax.broadcasted_iota(jnp.int32, (_NTP, _E), 1)
    totb = jnp.broadcast_to(total, (_NTP, _E))
    jc = jnp.minimum(j, totb - 1)
    hit = (jc >= jnp.broadcast_to(excl, (_NTP, _E))) & (
        jc < jnp.broadcast_to(excl + tilecnt, (_NTP, _E)))
    te_ref[...] = jnp.sum(jnp.where(hit, eb, 0), axis=1, keepdims=True)
    vf_ref[...] = (j[:, 0:1] < totb[:, 0:1]).astype(jnp.int32)


def _router(xf, wr, br):
    return pl.pallas_call(
        _router_body,
        out_shape=(
            jax.ShapeDtypeStruct((_T, 2), jnp.int32),
            jax.ShapeDtypeStruct((_T, 2), jnp.float32),
            jax.ShapeDtypeStruct((_NTP, 1), jnp.int32),
            jax.ShapeDtypeStruct((_NTP, 1), jnp.int32),
        ),
    )(xf, wr, br)


def _scatter_sc(xv, epos):
    mesh = plsc.VectorSubcoreMesh(core_axis_name="c", subcore_axis_name="s")

    @pl.kernel(out_type=jax.ShapeDtypeStruct((_P * _F, _DV), jnp.float32),
               mesh=mesh, scratch_types=[])
    def scatter_kernel(x_hbm, p0_hbm, p1_hbm, gx_hbm):
        def body(x_vmem, i0_vmem, i1_vmem):
            pltpu.sync_copy(x_vmem, gx_hbm.at[i0_vmem.at[0]])
            pltpu.sync_copy(x_vmem, gx_hbm.at[i1_vmem.at[0]])

        pltpu.emit_pipeline(
            body,
            grid=(_T * _F // _WSC,),
            in_specs=[
                pl.BlockSpec((_WSC, _DV), lambda i: (i, 0)),
                pl.BlockSpec((1, _WSC), lambda i: (0, i)),
                pl.BlockSpec((1, _WSC), lambda i: (1, i)),
            ],
            out_specs=[],
            core_axis_name=("c", "s"),
            dimension_semantics=(pltpu.PARALLEL,),
        )(x_hbm, p0_hbm, p1_hbm)

    return scatter_kernel(xv, epos, epos)


def _mlp_body(te_ref, vf_ref, gx_ref, w1_ref, b1_ref, w2_ref, b2_ref, out_ref):
    @pl.when(vf_ref[pl.program_id(0)] == 1)
    def _():
        a = gx_ref[...]
        h = lax.dot_general(
            a, w1_ref[0], (((1,), (0,)), ((), ())),
            preferred_element_type=jnp.float32) + b1_ref[0]
        h = jnp.maximum(h, 0.0)
        out_ref[...] = lax.dot_general(
            h, w2_ref[0], (((1,), (0,)), ((), ())),
            preferred_element_type=jnp.float32) + b2_ref[0]


def _mlp(te, vf, gx, W1, b1, W2, b2):
    grid_spec = pltpu.PrefetchScalarGridSpec(
        num_scalar_prefetch=2,
        grid=(_NT,),
        in_specs=[
            pl.BlockSpec((_TILE, _D), lambda i, te_r, vf_r: (i, 0)),
            pl.BlockSpec((1, _D, _H), lambda i, te_r, vf_r: (te_r[i], 0, 0)),
            pl.BlockSpec((1, 1, _H), lambda i, te_r, vf_r: (te_r[i], 0, 0)),
            pl.BlockSpec((1, _H, _D), lambda i, te_r, vf_r: (te_r[i], 0, 0)),
            pl.BlockSpec((1, 1, _D), lambda i, te_r, vf_r: (te_r[i], 0, 0)),
        ],
        out_specs=pl.BlockSpec((_TILE, _D), lambda i, te_r, vf_r: (i, 0)),
    )
    return pl.pallas_call(
        _mlp_body,
        grid_spec=grid_spec,
        out_shape=jax.ShapeDtypeStruct((_P, _D), jnp.float32),
    )(te, vf, gx, W1, b1, W2, b2)


def _gather_sc(eoutv, epos):
    mesh = plsc.VectorSubcoreMesh(core_axis_name="c", subcore_axis_name="s")
    nwin = _T * _F // _WSC

    @pl.kernel(out_type=jax.ShapeDtypeStruct((2 * _T * _F, _DV), jnp.float32),
               mesh=mesh, scratch_types=[])
    def gather_kernel(eout_hbm, p_hbm, g_hbm):
        def body(i_vmem, g_vmem):
            pltpu.sync_copy(eout_hbm.at[i_vmem.at[0]], g_vmem)

        pltpu.emit_pipeline(
            body,
            grid=(2, nwin),
            in_specs=[pl.BlockSpec((1, _WSC), lambda k, i: (k, i))],
            out_specs=[pl.BlockSpec(
                (_WSC, _DV), lambda k, i: (k * nwin + i, 0))],
            core_axis_name=("c", "s"),
            dimension_semantics=(pltpu.PARALLEL, pltpu.PARALLEL),
        )(p_hbm, g_hbm)

    return gather_kernel(eoutv, epos)


def _combine_body(g0_ref, g1_ref, wv_ref, out_ref):
    out_ref[...] = (g0_ref[...] * wv_ref[:, 0:1]
                    + g1_ref[...] * wv_ref[:, 1:2])


def _combine(g, wv):
    nblk = _T // _TILE
    return pl.pallas_call(
        _combine_body,
        grid=(nblk,),
        in_specs=[
            pl.BlockSpec((_TILE, _D), lambda i: (i, 0)),
            pl.BlockSpec((_TILE, _D), lambda i: (nblk + i, 0)),
            pl.BlockSpec((_TILE, 2), lambda i: (i, 0)),
        ],
        out_specs=pl.BlockSpec((_TILE, _D), lambda i: (i, 0)),
        out_shape=jax.ShapeDtypeStruct((_T, _D), jnp.float32),
    )(g, g, wv)


def kernel(x, W_route, b_route, W_noise, b_noise, W1, b1, W2, b2):
    del W_noise, b_noise
    xf = x.reshape(_T, _D)
    pos, wv, te, vf = _router(xf, W_route, b_route.reshape(1, _E))
    te = te.reshape(_NTP)
    vf = vf.reshape(_NTP)
    q = jnp.stack([_F * pos, _F * pos + 1], axis=2)
    epos = q.transpose(1, 0, 2).reshape(2, _T * _F)
    gxv = _scatter_sc(xf.reshape(_T * _F, _DV), epos)
    eout = _mlp(te, vf, gxv.reshape(_P, _D), W1, b1.reshape(_E, 1, _H), W2,
                b2.reshape(_E, 1, _D))
    gv = _gather_sc(eout.reshape(_P * _F, _DV), epos)
    out = _combine(gv.reshape(2 * _T, _D), wv)
    return out.reshape(1, _T, _D)


# speedup vs baseline: 1.2680x; 1.0250x over previous
import jax
import jax.numpy as jnp
from jax import lax
from jax.experimental import pallas as pl
from jax.experimental.pallas import tpu as pltpu
from jax.experimental.pallas import tpu_sc as plsc

_T = 2048
_D = 768
_E = 8
_H = 3072
_TILE = 512
_NT = 15
_NTP = 32
_P = _NT * _TILE
_WSC = 128
_F = 2
_DV = _D // _F


def _router_body(x_ref, wr_ref, br_ref, pos_ref, wv_ref, te_ref, vf_ref):
    x = x_ref[...]
    logits = lax.dot_general(
        x, wr_ref[...], (((1,), (0,)), ((), ())),
        preferred_element_type=jnp.float32) + br_ref[...]
    ie = lax.broadcasted_iota(jnp.int32, (_T, _E), 1)

    v1 = jnp.max(logits, axis=1, keepdims=True)
    i1 = jnp.min(jnp.where(logits == v1, ie, _E), axis=1, keepdims=True)
    ch1 = ie == i1
    l2 = jnp.where(ch1, -jnp.inf, logits)
    v2 = jnp.max(l2, axis=1, keepdims=True)
    i2 = jnp.min(jnp.where(l2 == v2, ie, _E), axis=1, keepdims=True)
    ch2 = ie == i2

    e2 = jnp.exp(v2 - v1)
    denom = 1.0 + e2
    w1 = 1.0 / denom
    w2 = e2 / denom

    m = ch1.astype(jnp.int32) + ch2.astype(jnp.int32)
    cs = m
    s = 1
    while s < _T:
        cs = cs + jnp.concatenate(
            [jnp.zeros((s, _E), jnp.int32), cs[:-s]], axis=0)
        s *= 2
    counts = cs[_T - 1:_T, :]
    tilecnt = (counts + _TILE - 1) // _TILE
    incl = tilecnt
    s = 1
    while s < _E:
        incl = incl + jnp.concatenate(
            [jnp.zeros((1, s), jnp.int32), incl[:, :-s]], axis=1)
        s *= 2
    excl = incl - tilecnt
    start = excl * _TILE
    total = incl[:, _E - 1:_E]

    cse = cs - m
    startb = jnp.broadcast_to(start, (_T, _E))
    p1 = (jnp.sum(jnp.where(ch1, cse, 0), axis=1, keepdims=True)
          + jnp.sum(jnp.where(ch1, startb, 0), axis=1, keepdims=True))
    p2 = (jnp.sum(jnp.where(ch2, cse, 0), axis=1, keepdims=True)
          + jnp.sum(jnp.where(ch2, startb, 0), axis=1, keepdims=True))
    pos_ref[...] = jnp.concatenate([p1, p2], axis=1)
    wv_ref[...] = jnp.concatenate([w1, w2], axis=1)

    j = lax.broadcasted_iota(jnp.int32, (_NTP, _E), 0)
    eb = lax.broadcasted_iota(jnp.int32, (_NTP, _E), 1)
    totb = jnp.broadcast_to(total, (_NTP, _E))
    jc = jnp.minimum(j, totb - 1)
    hit = (jc >= jnp.broadcast_to(excl, (_NTP, _E))) & (
        jc < jnp.broadcast_to(excl + tilecnt, (_NTP, _E)))
    te_ref[...] = jnp.sum(jnp.where(hit, eb, 0), axis=1, keepdims=True)
    vf_ref[...] = (j[:, 0:1] < totb[:, 0:1]).astype(jnp.int32)


def _router(xf, wr, br):
    return pl.pallas_call(
        _router_body,
        out_shape=(
            jax.ShapeDtypeStruct((_T, 2), jnp.int32),
            jax.ShapeDtypeStruct((_T, 2), jnp.float32),
            jax.ShapeDtypeStruct((_NTP, 1), jnp.int32),
            jax.ShapeDtypeStruct((_NTP, 1), jnp.int32),
        ),
    )(xf, wr, br)


def _scatter_sc(xv, epos):
    mesh = plsc.VectorSubcoreMesh(core_axis_name="c", subcore_axis_name="s")

    @pl.kernel(out_type=jax.ShapeDtypeStruct((_P * _F, _DV), jnp.float32),
               mesh=mesh, scratch_types=[])
    def scatter_kernel(x_hbm, p0_hbm, p1_hbm, gx_hbm):
        def body(x_vmem, i0_vmem, i1_vmem):
            pltpu.sync_copy(x_vmem, gx_hbm.at[i0_vmem.at[0]])
            pltpu.sync_copy(x_vmem, gx_hbm.at[i1_vmem.at[0]])

        pltpu.emit_pipeline(
            body,
            grid=(_T * _F // _WSC,),
            in_specs=[
                pl.BlockSpec((_WSC, _DV), lambda i: (i, 0)),
                pl.BlockSpec((1, _WSC), lambda i: (0, i)),
                pl.BlockSpec((1, _WSC), lambda i: (1, i)),
            ],
            out_specs=[],
            core_axis_name=("c", "s"),
            dimension_semantics=(pltpu.PARALLEL,),
        )(x_hbm, p0_hbm, p1_hbm)

    return scatter_kernel(xv, epos, epos)


def _mlp_body(te_ref, vf_ref, gx_ref, w1_ref, b1_ref, w2_ref, b2_ref, out_ref):
    @pl.when(vf_ref[pl.program_id(0)] == 1)
    def _():
        a = gx_ref[...]
        h = lax.dot_general(
            a, w1_ref[0], (((1,), (0,)), ((), ())),
            preferred_element_type=jnp.float32) + b1_ref[0]
        h = jnp.maximum(h, 0.0)
        out_ref[...] = lax.dot_general(
            h, w2_ref[0], (((1,), (0,)), ((), ())),
            preferred_element_type=jnp.float32) + b2_ref[0]


def _mlp(te, vf, gx, W1, b1, W2, b2):
    grid_spec = pltpu.PrefetchScalarGridSpec(
        num_scalar_prefetch=2,
        grid=(_NT,),
        in_specs=[
            pl.BlockSpec((_TILE, _D), lambda i, te_r, vf_r: (i, 0)),
            pl.BlockSpec((1, _D, _H), lambda i, te_r, vf_r: (te_r[i], 0, 0)),
            pl.BlockSpec((1, 1, _H), lambda i, te_r, vf_r: (te_r[i], 0, 0)),
            pl.BlockSpec((1, _H, _D), lambda i, te_r, vf_r: (te_r[i], 0, 0)),
            pl.BlockSpec((1, 1, _D), lambda i, te_r, vf_r: (te_r[i], 0, 0)),
        ],
        out_specs=pl.BlockSpec((_TILE, _D), lambda i, te_r, vf_r: (i, 0)),
    )
    return pl.pallas_call(
        _mlp_body,
        grid_spec=grid_spec,
        out_shape=jax.ShapeDtypeStruct((_P, _D), jnp.float32),
    )(te, vf, gx, W1, b1, W2, b2)


def _gather_sc(eoutv, epos):
    mesh = plsc.VectorSubcoreMesh(core_axis_name="c", subcore_axis_name="s")
    nwin = _T * _F // _WSC

    @pl.kernel(out_type=jax.ShapeDtypeStruct((2 * _T * _F, _DV), jnp.float32),
               mesh=mesh, scratch_types=[])
    def gather_kernel(eout_hbm, p_hbm, g_hbm):
        def body(i_vmem, g_vmem):
            pltpu.sync_copy(eout_hbm.at[i_vmem.at[0]], g_vmem)

        pltpu.emit_pipeline(
            body,
            grid=(2, nwin),
            in_specs=[pl.BlockSpec((1, _WSC), lambda k, i: (k, i))],
            out_specs=[pl.BlockSpec(
                (_WSC, _DV), lambda k, i: (k * nwin + i, 0))],
            core_axis_name=("c", "s"),
            dimension_semantics=(pltpu.PARALLEL, pltpu.PARALLEL),
        )(p_hbm, g_hbm)

    return gather_kernel(eoutv, epos)


def _combine_body(g0_ref, g1_ref, wv_ref, out_ref):
    out_ref[...] = (g0_ref[...] * wv_ref[:, 0:1]
                    + g1_ref[...] * wv_ref[:, 1:2])


def _combine(g, wv):
    nblk = _T // _TILE
    return pl.pallas_call(
        _combine_body,
        grid=(nblk,),
        in_specs=[
            pl.BlockSpec((_TILE, _D), lambda i: (i, 0)),
            pl.BlockSpec((_TILE, _D), lambda i: (nblk + i, 0)),
            pl.BlockSpec((_TILE, 2), lambda i: (i, 0)),
        ],
        out_specs=pl.BlockSpec((_TILE, _D), lambda i: (i, 0)),
        out_shape=jax.ShapeDtypeStruct((_T, _D), jnp.float32),
    )(g, g, wv)


def kernel(x, W_route, b_route, W_noise, b_noise, W1, b1, W2, b2):
    del W_noise, b_noise
    xf = x.reshape(_T, _D)
    pos, wv, te, vf = _router(xf, W_route, b_route.reshape(1, _E))
    te = te.reshape(_NTP)
    vf = vf.reshape(_NTP)
    q = jnp.stack([_F * pos, _F * pos + 1], axis=2)
    epos = q.transpose(1, 0, 2).reshape(2, _T * _F)
    gxv = _scatter_sc(xf.reshape(_T * _F, _DV), epos)
    eout = _mlp(te, vf, gxv.reshape(_P, _D), W1, b1.reshape(_E, 1, _H), W2,
                b2.reshape(_E, 1, _D))
    gv = _gather_sc(eout.reshape(_P * _F, _DV), epos)
    out = _combine(gv.reshape(2 * _T, _D), wv)
    return out.reshape(1, _T, _D)
